# baseline (speedup 1.0000x reference)
import functools

import jax
import jax.numpy as jnp
from jax import lax
from jax.experimental import pallas as pl
from jax.experimental.pallas import tpu as pltpu
from jax.experimental.pallas import tpu_sc as plsc

NC = 2
NS = 16
NW = NC * NS

PAD_D = 128
CHUNK = 128
GATHERS_PER_SLAB = 2
SLAB = CHUNK * GATHERS_PER_SLAB


def _emb_call(n_flat):
    mesh = plsc.VectorSubcoreMesh(core_axis_name="c", subcore_axis_name="s")
    per_w = n_flat // NW
    n_slabs = per_w // SLAB

    @functools.partial(
        pl.kernel,
        mesh=mesh,
        out_type=jax.ShapeDtypeStruct((n_flat, PAD_D), jnp.float32),
        scratch_types=[
            pltpu.VMEM((per_w,), jnp.int32),
            pltpu.VMEM((2, SLAB, PAD_D), jnp.float32),
            pltpu.SemaphoreType.DMA,
            pltpu.SemaphoreType.DMA,
            pltpu.SemaphoreType.DMA,
            pltpu.SemaphoreType.DMA,
        ],
        compiler_params=pltpu.CompilerParams(use_tc_tiling_on_sc=True),
    )
    def emb(idx_hbm, table_hbm, out_hbm, idx_v, rows_v, g_sem0, g_sem1,
            s_sem0, s_sem1):
        wid = lax.axis_index("s") * NC + lax.axis_index("c")
        base = wid * per_w
        pltpu.sync_copy(idx_hbm.at[pl.ds(base, per_w)], idx_v)

        def fire(g, b, g_sem):
            for j in range(GATHERS_PER_SLAB):
                pltpu.async_copy(
                    table_hbm.at[idx_v.at[pl.ds(g * SLAB + j * CHUNK, CHUNK)]],
                    rows_v.at[b].at[pl.ds(j * CHUNK, CHUNK)],
                    g_sem,
                )

        def drain(b, g_sem):
            for j in range(GATHERS_PER_SLAB):
                pltpu.make_async_copy(
                    table_hbm.at[pl.ds(0, CHUNK)],
                    rows_v.at[b].at[pl.ds(j * CHUNK, CHUNK)],
                    g_sem,
                ).wait()

        def store(g, b, s_sem):
            pltpu.async_copy(
                rows_v.at[b], out_hbm.at[pl.ds(base + g * SLAB, SLAB)], s_sem
            )

        def wait_store(b, s_sem):
            pltpu.make_async_copy(
                out_hbm.at[pl.ds(0, SLAB)], rows_v.at[b], s_sem
            ).wait()

        fire(0, 0, g_sem0)

        def body(g, carry):
            b = lax.rem(g, 2)

            @pl.when(b == 0)
            def _even():
                @pl.when(g >= 1)
                def _():
                    wait_store(1, s_sem1)
                fire(g + 1, 1, g_sem1)
                drain(0, g_sem0)
                store(g, 0, s_sem0)

            @pl.when(b == 1)
            def _odd():
                wait_store(0, s_sem0)
                fire(g + 1, 0, g_sem0)
                drain(1, g_sem1)
                store(g, 1, s_sem1)

            return carry

        lax.fori_loop(0, n_slabs - 1, body, 0)

        g_last = n_slabs - 1
        b_last = g_last % 2
        g_sem_last = g_sem0 if b_last == 0 else g_sem1
        s_sem_last = s_sem0 if b_last == 0 else s_sem1
        drain(b_last, g_sem_last)
        if n_slabs > 1:
            b_prev = 1 - b_last
            wait_store(b_prev, s_sem0 if b_prev == 0 else s_sem1)
        store(g_last, b_last, s_sem_last)
        wait_store(b_last, s_sem_last)

    return emb


def kernel(token_ids, weight):
    n_tok, s = token_ids.shape
    n, d = weight.shape
    tok1 = token_ids.reshape(-1)
    w_pad = jnp.pad(weight, ((0, 0), (0, PAD_D - d)))
    out_pad = _emb_call(n_tok * s)(tok1, w_pad)
    return out_pad[:, :d].reshape(n_tok, s, d)


# speedup vs baseline: 1.3553x; 1.3553x over previous
import functools

import jax
import jax.numpy as jnp
from jax import lax
from jax.experimental import pallas as pl
from jax.experimental.pallas import tpu as pltpu
from jax.experimental.pallas import tpu_sc as plsc

NC = 2
NS = 16
NW = NC * NS

PAD_D = 128
SLAB = 128
PACK = 4
OUT_ROWS = SLAB // PACK
L = 16


def _emb_call(n_flat, d):
    mesh = plsc.VectorSubcoreMesh(core_axis_name="c", subcore_axis_name="s")
    per_w = n_flat // NW
    n_slabs = per_w // SLAB

    @functools.partial(
        pl.kernel,
        mesh=mesh,
        out_type=jax.ShapeDtypeStruct((n_flat // PACK, PAD_D), jnp.float32),
        scratch_types=[
            pltpu.VMEM((per_w,), jnp.int32),
            pltpu.VMEM((2, SLAB, PAD_D), jnp.float32),
            pltpu.VMEM((2, OUT_ROWS, PAD_D), jnp.float32),
            pltpu.SemaphoreType.DMA,
            pltpu.SemaphoreType.DMA,
            pltpu.SemaphoreType.DMA,
            pltpu.SemaphoreType.DMA,
        ],
        compiler_params=pltpu.CompilerParams(use_tc_tiling_on_sc=True),
    )
    def emb(idx_hbm, table_hbm, out_hbm, idx_v, rows_v, cmp_v, g_sem0,
            g_sem1, s_sem0, s_sem1):
        wid = lax.axis_index("s") * NC + lax.axis_index("c")
        base = wid * per_w
        base_out = wid * (per_w // PACK)
        pltpu.sync_copy(idx_hbm.at[pl.ds(base, per_w)], idx_v)

        def fire(g, b, g_sem):
            pltpu.async_copy(
                table_hbm.at[idx_v.at[pl.ds(g * SLAB, SLAB)]],
                rows_v.at[b],
                g_sem,
            )

        def drain(b, g_sem):
            pltpu.make_async_copy(
                table_hbm.at[pl.ds(0, SLAB)], rows_v.at[b], g_sem
            ).wait()

        def compact(b):
            def cbody(i, carry):
                for u in range(8):
                    row = 2 * i + (u // 4)
                    for k in range(2):
                        v = rows_v[b, i * 8 + u, pl.ds(k * L, L)]
                        cmp_v[b, row, pl.ds((u % 4) * d + k * L, L)] = v
                return carry

            lax.fori_loop(0, SLAB // 8, cbody, 0)

        def store(g, b, s_sem):
            pltpu.async_copy(
                cmp_v.at[b],
                out_hbm.at[pl.ds(base_out + g * OUT_ROWS, OUT_ROWS)],
                s_sem,
            )

        def wait_store(b, s_sem):
            pltpu.make_async_copy(
                out_hbm.at[pl.ds(0, OUT_ROWS)], cmp_v.at[b], s_sem
            ).wait()

        fire(0, 0, g_sem0)

        def body(g, carry):
            b = lax.rem(g, 2)

            @pl.when(b == 0)
            def _even():
                fire(g + 1, 1, g_sem1)
                drain(0, g_sem0)

                @pl.when(g >= 2)
                def _():
                    wait_store(0, s_sem0)
                compact(0)
                store(g, 0, s_sem0)

            @pl.when(b == 1)
            def _odd():
                fire(g + 1, 0, g_sem0)
                drain(1, g_sem1)

                @pl.when(g >= 2)
                def _():
                    wait_store(1, s_sem1)
                compact(1)
                store(g, 1, s_sem1)

            return carry

        lax.fori_loop(0, n_slabs - 1, body, 0)

        g_last = n_slabs - 1
        b_last = g_last % 2
        g_sem_last = g_sem0 if b_last == 0 else g_sem1
        s_sem_last = s_sem0 if b_last == 0 else s_sem1
        drain(b_last, g_sem_last)
        if n_slabs > 2:
            wait_store(b_last, s_sem_last)
        compact(b_last)
        store(g_last, b_last, s_sem_last)
        if n_slabs > 1:
            b_prev = 1 - b_last
            wait_store(b_prev, s_sem0 if b_prev == 0 else s_sem1)
        wait_store(b_last, s_sem_last)

    return emb


def kernel(token_ids, weight):
    n_tok, s = token_ids.shape
    n, d = weight.shape
    tok1 = token_ids.reshape(-1)
    w_pad = jnp.pad(weight, ((0, 0), (0, PAD_D - d)))
    out_packed = _emb_call(n_tok * s, d)(tok1, w_pad)
    return out_packed.reshape(n_tok, s, d)


# speedup vs baseline: 1.4940x; 1.1023x over previous
import functools

import jax
import jax.numpy as jnp
from jax import lax
from jax.experimental import pallas as pl
from jax.experimental.pallas import tpu as pltpu
from jax.experimental.pallas import tpu_sc as plsc

NC = 2
NS = 16
NW = NC * NS

PAD_D = 128
ROWS_PER_SLAB = 4
L = 16


def _emb_call(n_tok, s, d):
    mesh = plsc.VectorSubcoreMesh(core_axis_name="c", subcore_axis_name="s")
    rows_per_w = n_tok // NW
    slab_tok = ROWS_PER_SLAB * s
    per_w = rows_per_w * s
    n_slabs = rows_per_w // ROWS_PER_SLAB
    g_sizes = [128, slab_tok - 128]

    @functools.partial(
        pl.kernel,
        mesh=mesh,
        out_type=jax.ShapeDtypeStruct((n_tok, s, d), jnp.float32),
        scratch_types=[
            pltpu.VMEM((per_w,), jnp.int32),
            pltpu.VMEM((2, slab_tok, PAD_D), jnp.float32),
            pltpu.VMEM((ROWS_PER_SLAB, s, d), jnp.float32),
            pltpu.SemaphoreType.DMA,
            pltpu.SemaphoreType.DMA,
            pltpu.SemaphoreType.DMA,
        ],
        compiler_params=pltpu.CompilerParams(use_tc_tiling_on_sc=True),
    )
    def emb(idx_hbm, table_hbm, out_hbm, idx_v, rows_v, cmp_v, g_sem0,
            g_sem1, s_sem):
        wid = lax.axis_index("s") * NC + lax.axis_index("c")
        base_r = wid * rows_per_w
        pltpu.sync_copy(idx_hbm.at[pl.ds(wid * per_w, per_w)], idx_v)

        def fire(g, b, g_sem):
            off = 0
            for sz in g_sizes:
                pltpu.async_copy(
                    table_hbm.at[idx_v.at[pl.ds(g * slab_tok + off, sz)]],
                    rows_v.at[b].at[pl.ds(off, sz)],
                    g_sem,
                )
                off += sz

        def drain(b, g_sem):
            off = 0
            for sz in g_sizes:
                pltpu.make_async_copy(
                    table_hbm.at[pl.ds(0, sz)],
                    rows_v.at[b].at[pl.ds(off, sz)],
                    g_sem,
                ).wait()
                off += sz

        def compact(b):
            for r in range(ROWS_PER_SLAB):
                def cbody(i, carry, r=r):
                    for u in range(10):
                        sp = i * 10 + u
                        t = r * s + sp
                        for k in range(d // L):
                            cmp_v[r, sp, pl.ds(k * L, L)] = (
                                rows_v[b, t, pl.ds(k * L, L)]
                            )
                    return carry

                lax.fori_loop(0, s // 10, cbody, 0)

        def store(g, s_sem):
            pltpu.async_copy(
                cmp_v,
                out_hbm.at[pl.ds(base_r + g * ROWS_PER_SLAB, ROWS_PER_SLAB)],
                s_sem,
            )

        def wait_store(s_sem):
            pltpu.make_async_copy(
                out_hbm.at[pl.ds(0, ROWS_PER_SLAB)], cmp_v, s_sem
            ).wait()

        fire(0, 0, g_sem0)

        def body(g, carry):
            parity = lax.rem(g, 2)

            @pl.when(parity == 0)
            def _even():
                fire(g + 1, 1, g_sem1)
                drain(0, g_sem0)

                @pl.when(g >= 1)
                def _():
                    wait_store(s_sem)
                compact(0)
                store(g, s_sem)

            @pl.when(parity == 1)
            def _odd():
                fire(g + 1, 0, g_sem0)
                drain(1, g_sem1)
                wait_store(s_sem)
                compact(1)
                store(g, s_sem)

            return carry

        lax.fori_loop(0, n_slabs - 1, body, 0)

        g_last = n_slabs - 1
        b_last = g_last % 2
        drain(b_last, g_sem0 if b_last == 0 else g_sem1)
        if n_slabs > 1:
            wait_store(s_sem)
        compact(b_last)
        store(g_last, s_sem)
        wait_store(s_sem)

    return emb


def kernel(token_ids, weight):
    n_tok, s = token_ids.shape
    n, d = weight.shape
    tok1 = token_ids.reshape(-1)
    w_pad = jnp.pad(weight, ((0, 0), (0, PAD_D - d)))
    return _emb_call(n_tok, s, d)(tok1, w_pad)
